# baseline (speedup 1.0000x reference)
import functools

import jax
import jax.numpy as jnp
from jax import lax
from jax.experimental import pallas as pl
from jax.experimental.pallas import tpu as pltpu
from jax.experimental.pallas import tpu_sc as plsc

T = 2048
D = 2048
F = 1024
E = 8
K = 2
TILE = 128
A = T * K
P = A + E * TILE
NT = P // TILE
CB = 128
NCB = A // CB

NC = 2
NS = 16
NW = NC * NS
CHS = 16
APW = A // NW
NCH = APW // CHS


def _fiota(shape, dim):
    return lax.broadcasted_iota(jnp.int32, shape, dim).astype(jnp.float32)


def _router_kernel(x_ref, rw_ref, pos_ref, texp_ref, gates_ref, aux_ref):
    xf = x_ref[...]
    logits = jax.lax.dot_general(
        xf, rw_ref[...], (((1,), (1,)), ((), ())),
        preferred_element_type=jnp.float32)
    m = jnp.max(logits, axis=1, keepdims=True)
    ex = jnp.exp(logits - m)
    probs = ex / jnp.sum(ex, axis=1, keepdims=True)

    eidx = _fiota((T, E), 1)
    v0 = jnp.max(probs, axis=1, keepdims=True)
    i0 = jnp.min(jnp.where(probs == v0, eidx, float(E)), axis=1,
                 keepdims=True)
    probs2 = jnp.where(eidx == i0, -1.0, probs)
    v1 = jnp.max(probs2, axis=1, keepdims=True)
    i1 = jnp.min(jnp.where(probs2 == v1, eidx, float(E)), axis=1,
                 keepdims=True)
    s = v0 + v1
    gates_ref[...] = jnp.concatenate([v0 / s, v1 / s], axis=1)

    M0 = (eidx == i0).astype(jnp.float32)
    M1 = (eidx == i1).astype(jnp.float32)
    rT = _fiota((T, T), 0)
    cT = _fiota((T, T), 1)
    LT = (cT < rT).astype(jnp.float32)
    Mcat = jnp.concatenate([M0, M1], axis=1)
    cscat = jax.lax.dot_general(LT, Mcat, (((1,), (0,)), ((), ())),
                                preferred_element_type=jnp.float32)
    cs0 = cscat[:, :E]
    cs1 = cscat[:, E:]

    count0 = jnp.sum(M0, axis=0, keepdims=True)
    counts = count0 + jnp.sum(M1, axis=0, keepdims=True)
    ac = jnp.ceil(counts / TILE) * TILE
    r8 = _fiota((E, E), 0)
    c8 = _fiota((E, E), 1)
    S8 = (r8 < c8).astype(jnp.float32)
    off = jax.lax.dot_general(ac, S8, (((1,), (0,)), ((), ())),
                              preferred_element_type=jnp.float32)

    pos0 = jnp.sum(M0 * (off + cs0), axis=1, keepdims=True)
    pos1 = jnp.sum(M1 * (off + count0 + cs1), axis=1, keepdims=True)
    pos_ref[...] = jnp.concatenate([pos0, pos1], axis=1).astype(jnp.int32)

    tstart = _fiota((NT, E), 0) * float(TILE)
    te = jnp.sum((jnp.broadcast_to(off, (NT, E)) <= tstart)
                 .astype(jnp.float32), axis=1, keepdims=True) - 1.0
    texp_ref[...] = te.astype(jnp.int32)

    f = counts / float(T)
    pmean = jnp.mean(probs, axis=0, keepdims=True)
    aux_ref[...] = jnp.sum(f * pmean, axis=1, keepdims=True) * (float(E) * 0.01)


def _router(xf, router_w):
    return pl.pallas_call(
        _router_kernel,
        out_shape=(
            jax.ShapeDtypeStruct((T, 2), jnp.int32),
            jax.ShapeDtypeStruct((NT, 1), jnp.int32),
            jax.ShapeDtypeStruct((T, 2), jnp.float32),
            jax.ShapeDtypeStruct((1, 1), jnp.float32),
        ),
    )(xf, router_w)


@functools.lru_cache(maxsize=None)
def _sc_kernels():
    mesh = plsc.VectorSubcoreMesh(
        core_axis_name="c", subcore_axis_name="s",
        num_cores=NC, num_subcores=NS)
    scratch = [
        pltpu.VMEM((NCH, CHS), jnp.int32),
        pltpu.VMEM((CHS, D), jnp.float32),
        pltpu.SemaphoreType.DMA,
    ]

    @functools.partial(
        pl.kernel,
        out_type=jax.ShapeDtypeStruct((P, D), jnp.float32),
        mesh=mesh, scratch_types=scratch)
    def sc_scatter_x(x_hbm, pos_hbm, xg_hbm, idx_v, rows_v, sem):
        wid = lax.axis_index("s") * NC + lax.axis_index("c")
        pltpu.sync_copy(pos_hbm.at[wid], idx_v)
        t0 = (wid * APW) % T

        def body(j, _):
            pltpu.sync_copy(x_hbm.at[pl.ds(t0 + j * CHS, CHS)], rows_v)
            pltpu.async_copy(rows_v, xg_hbm.at[idx_v.at[j]], sem).wait()
            return 0

        lax.fori_loop(0, NCH, body, 0, unroll=True)

    @functools.partial(
        pl.kernel,
        out_type=jax.ShapeDtypeStruct((A, D), jnp.float32),
        mesh=mesh, scratch_types=scratch)
    def sc_gather_y(yg_hbm, pos_hbm, yc_hbm, idx_v, rows_v, sem):
        wid = lax.axis_index("s") * NC + lax.axis_index("c")
        pltpu.sync_copy(pos_hbm.at[wid], idx_v)
        base = wid * APW

        def body(j, _):
            pltpu.async_copy(yg_hbm.at[idx_v.at[j]], rows_v, sem).wait()
            pltpu.sync_copy(rows_v, yc_hbm.at[pl.ds(base + j * CHS, CHS)])
            return 0

        lax.fori_loop(0, NCH, body, 0, unroll=True)

    return sc_scatter_x, sc_gather_y


def _group_kernel(texp_ref, xg_ref, wfc_ref, wproj_ref, yg_ref):
    rows = xg_ref[...].astype(jnp.bfloat16)
    h = jax.lax.dot_general(rows, wfc_ref[0].astype(jnp.bfloat16),
                            (((1,), (1,)), ((), ())),
                            preferred_element_type=jnp.float32)
    g = h[:, :F]
    u = h[:, F:]
    act = (g * jax.nn.sigmoid(g) * u).astype(jnp.bfloat16)
    yg_ref[...] = jax.lax.dot_general(
        act, wproj_ref[0].astype(jnp.bfloat16), (((1,), (1,)), ((), ())),
        preferred_element_type=jnp.float32)


def _grouped_mlp(xg, w_fc, w_proj, tile_expert):
    grid_spec = pltpu.PrefetchScalarGridSpec(
        num_scalar_prefetch=1,
        grid=(NT,),
        in_specs=[
            pl.BlockSpec((TILE, D), lambda i, te: (i, 0)),
            pl.BlockSpec((1, 2 * F, D), lambda i, te: (te[i], 0, 0)),
            pl.BlockSpec((1, D, F), lambda i, te: (te[i], 0, 0)),
        ],
        out_specs=pl.BlockSpec((TILE, D), lambda i, te: (i, 0)),
    )
    return pl.pallas_call(
        _group_kernel,
        grid_spec=grid_spec,
        out_shape=jax.ShapeDtypeStruct((P, D), jnp.float32),
    )(tile_expert, xg, w_fc, w_proj)


def _combine_kernel(y0_ref, y1_ref, gates_ref, out_ref):
    gts = gates_ref[...]
    out_ref[...] = (y0_ref[...] * gts[:, 0:1] + y1_ref[...] * gts[:, 1:2])


def _combine(y0, y1, gates):
    blk = 256
    return pl.pallas_call(
        _combine_kernel,
        grid=(T // blk,),
        in_specs=[
            pl.BlockSpec((blk, D), lambda i: (i, 0)),
            pl.BlockSpec((blk, D), lambda i: (i, 0)),
            pl.BlockSpec((blk, 2), lambda i: (i, 0)),
        ],
        out_specs=pl.BlockSpec((blk, D), lambda i: (i, 0)),
        out_shape=jax.ShapeDtypeStruct((T, D), jnp.float32),
    )(y0, y1, gates)


def kernel(x, router_w, w_fc, w_proj):
    b, s, d = x.shape
    xf = x.reshape(b * s, d)
    pos, tile_expert, gates, aux = _router(xf, router_w)

    sc_scatter_x, sc_gather_y = _sc_kernels()
    pos3 = pos.T.reshape(NW, NCH, CHS)
    xg = sc_scatter_x(xf, pos3)
    yg = _grouped_mlp(xg, w_fc, w_proj, tile_expert[:, 0])
    yc = sc_gather_y(yg, pos3)
    y = _combine(yc[:T], yc[T:], gates).reshape(b, s, d)
    return (y, aux[0, 0])


# speedup vs baseline: 1.0697x; 1.0697x over previous
import functools

import jax
import jax.numpy as jnp
from jax import lax
from jax.experimental import pallas as pl
from jax.experimental.pallas import tpu as pltpu
from jax.experimental.pallas import tpu_sc as plsc

T = 2048
D = 2048
F = 1024
E = 8
K = 2
TILE = 512
A = T * K
P = A + E * TILE
NT = P // TILE
CB = 128
NCB = A // CB

NC = 2
NS = 16
NW = NC * NS
CHS = 16
APW = A // NW
NCH = APW // CHS


def _fiota(shape, dim):
    return lax.broadcasted_iota(jnp.int32, shape, dim).astype(jnp.float32)


def _router_kernel(x_ref, rw_ref, pos_ref, texp_ref, gates_ref, aux_ref):
    xf = x_ref[...]
    logits = jax.lax.dot_general(
        xf, rw_ref[...], (((1,), (1,)), ((), ())),
        preferred_element_type=jnp.float32)
    m = jnp.max(logits, axis=1, keepdims=True)
    ex = jnp.exp(logits - m)
    probs = ex / jnp.sum(ex, axis=1, keepdims=True)

    eidx = _fiota((T, E), 1)
    v0 = jnp.max(probs, axis=1, keepdims=True)
    i0 = jnp.min(jnp.where(probs == v0, eidx, float(E)), axis=1,
                 keepdims=True)
    probs2 = jnp.where(eidx == i0, -1.0, probs)
    v1 = jnp.max(probs2, axis=1, keepdims=True)
    i1 = jnp.min(jnp.where(probs2 == v1, eidx, float(E)), axis=1,
                 keepdims=True)
    s = v0 + v1
    gates_ref[...] = jnp.concatenate([v0 / s, v1 / s], axis=1)

    M0 = (eidx == i0).astype(jnp.float32)
    M1 = (eidx == i1).astype(jnp.float32)
    rT = _fiota((T, T), 0)
    cT = _fiota((T, T), 1)
    LT = (cT < rT).astype(jnp.float32)
    Mcat = jnp.concatenate([M0, M1], axis=1)
    cscat = jax.lax.dot_general(LT, Mcat, (((1,), (0,)), ((), ())),
                                preferred_element_type=jnp.float32)
    cs0 = cscat[:, :E]
    cs1 = cscat[:, E:]

    count0 = jnp.sum(M0, axis=0, keepdims=True)
    counts = count0 + jnp.sum(M1, axis=0, keepdims=True)
    ac = jnp.ceil(counts / TILE) * TILE
    r8 = _fiota((E, E), 0)
    c8 = _fiota((E, E), 1)
    S8 = (r8 < c8).astype(jnp.float32)
    off = jax.lax.dot_general(ac, S8, (((1,), (0,)), ((), ())),
                              preferred_element_type=jnp.float32)

    pos0 = jnp.sum(M0 * (off + cs0), axis=1, keepdims=True)
    pos1 = jnp.sum(M1 * (off + count0 + cs1), axis=1, keepdims=True)
    pos_ref[...] = jnp.concatenate([pos0, pos1], axis=1).astype(jnp.int32)

    tstart = _fiota((NT, E), 0) * float(TILE)
    te = jnp.sum((jnp.broadcast_to(off, (NT, E)) <= tstart)
                 .astype(jnp.float32), axis=1, keepdims=True) - 1.0
    texp_ref[...] = te.astype(jnp.int32)

    f = counts / float(T)
    pmean = jnp.mean(probs, axis=0, keepdims=True)
    aux_ref[...] = jnp.sum(f * pmean, axis=1, keepdims=True) * (float(E) * 0.01)


def _router(xf, router_w):
    return pl.pallas_call(
        _router_kernel,
        out_shape=(
            jax.ShapeDtypeStruct((T, 2), jnp.int32),
            jax.ShapeDtypeStruct((NT, 1), jnp.int32),
            jax.ShapeDtypeStruct((T, 2), jnp.float32),
            jax.ShapeDtypeStruct((1, 1), jnp.float32),
        ),
    )(xf, router_w)


@functools.lru_cache(maxsize=None)
def _sc_kernels():
    mesh = plsc.VectorSubcoreMesh(
        core_axis_name="c", subcore_axis_name="s",
        num_cores=NC, num_subcores=NS)
    scratch = [
        pltpu.VMEM((NCH, CHS), jnp.int32),
        pltpu.VMEM((CHS, D), jnp.float32),
        pltpu.SemaphoreType.DMA,
    ]

    @functools.partial(
        pl.kernel,
        out_type=jax.ShapeDtypeStruct((P, D), jnp.float32),
        mesh=mesh, scratch_types=scratch)
    def sc_scatter_x(x_hbm, pos_hbm, xg_hbm, idx_v, rows_v, sem):
        wid = lax.axis_index("s") * NC + lax.axis_index("c")
        pltpu.sync_copy(pos_hbm.at[wid], idx_v)
        t0 = (wid * APW) % T

        def body(j, _):
            pltpu.sync_copy(x_hbm.at[pl.ds(t0 + j * CHS, CHS)], rows_v)
            pltpu.async_copy(rows_v, xg_hbm.at[idx_v.at[j]], sem).wait()
            return 0

        lax.fori_loop(0, NCH, body, 0, unroll=True)

    @functools.partial(
        pl.kernel,
        out_type=jax.ShapeDtypeStruct((A, D), jnp.float32),
        mesh=mesh, scratch_types=scratch)
    def sc_gather_y(yg_hbm, pos_hbm, yc_hbm, idx_v, rows_v, sem):
        wid = lax.axis_index("s") * NC + lax.axis_index("c")
        pltpu.sync_copy(pos_hbm.at[wid], idx_v)
        base = wid * APW

        def body(j, _):
            pltpu.async_copy(yg_hbm.at[idx_v.at[j]], rows_v, sem).wait()
            pltpu.sync_copy(rows_v, yc_hbm.at[pl.ds(base + j * CHS, CHS)])
            return 0

        lax.fori_loop(0, NCH, body, 0, unroll=True)

    return sc_scatter_x, sc_gather_y


def _group_kernel(texp_ref, xg_ref, wfc_ref, wproj_ref, yg_ref):
    rows = xg_ref[...].astype(jnp.bfloat16)
    h = jax.lax.dot_general(rows, wfc_ref[0], (((1,), (1,)), ((), ())),
                            preferred_element_type=jnp.float32)
    g = h[:, :F]
    u = h[:, F:]
    act = (g * jax.nn.sigmoid(g) * u).astype(jnp.bfloat16)
    yg_ref[...] = jax.lax.dot_general(
        act, wproj_ref[0], (((1,), (1,)), ((), ())),
        preferred_element_type=jnp.float32)


def _grouped_mlp(xg, w_fc, w_proj, tile_expert):
    grid_spec = pltpu.PrefetchScalarGridSpec(
        num_scalar_prefetch=1,
        grid=(NT,),
        in_specs=[
            pl.BlockSpec((TILE, D), lambda i, te: (i, 0)),
            pl.BlockSpec((1, 2 * F, D), lambda i, te: (te[i], 0, 0)),
            pl.BlockSpec((1, D, F), lambda i, te: (te[i], 0, 0)),
        ],
        out_specs=pl.BlockSpec((TILE, D), lambda i, te: (i, 0)),
    )
    return pl.pallas_call(
        _group_kernel,
        grid_spec=grid_spec,
        out_shape=jax.ShapeDtypeStruct((P, D), jnp.float32),
    )(tile_expert, xg, w_fc, w_proj)


def _combine_kernel(y0_ref, y1_ref, gates_ref, out_ref):
    gts = gates_ref[...]
    out_ref[...] = (y0_ref[...] * gts[:, 0:1] + y1_ref[...] * gts[:, 1:2])


def _combine(y0, y1, gates):
    blk = 256
    return pl.pallas_call(
        _combine_kernel,
        grid=(T // blk,),
        in_specs=[
            pl.BlockSpec((blk, D), lambda i: (i, 0)),
            pl.BlockSpec((blk, D), lambda i: (i, 0)),
            pl.BlockSpec((blk, 2), lambda i: (i, 0)),
        ],
        out_specs=pl.BlockSpec((blk, D), lambda i: (i, 0)),
        out_shape=jax.ShapeDtypeStruct((T, D), jnp.float32),
    )(y0, y1, gates)


def kernel(x, router_w, w_fc, w_proj):
    b, s, d = x.shape
    xf = x.reshape(b * s, d)
    pos, tile_expert, gates, aux = _router(xf, router_w)

    sc_scatter_x, sc_gather_y = _sc_kernels()
    pos3 = pos.T.reshape(NW, NCH, CHS)
    xg = sc_scatter_x(xf, pos3)
    yg = _grouped_mlp(xg, w_fc.astype(jnp.bfloat16),
                      w_proj.astype(jnp.bfloat16), tile_expert[:, 0])
    yc = sc_gather_y(yg, pos3)
    y = _combine(yc[:T], yc[T:], gates).reshape(b, s, d)
    return (y, aux[0, 0])


# speedup vs baseline: 1.3970x; 1.3060x over previous
import functools

import jax
import jax.numpy as jnp
from jax import lax
from jax.experimental import pallas as pl
from jax.experimental.pallas import tpu as pltpu
from jax.experimental.pallas import tpu_sc as plsc

T = 2048
D = 2048
F = 1024
E = 8
K = 2
TILE = 256
A = T * K
P = A + E * TILE
NT = P // TILE
CB = 128
NCB = A // CB

NC = 2
NS = 16
NW = NC * NS
CHS = 16
APW = A // NW
NCH = APW // CHS


def _fiota(shape, dim):
    return lax.broadcasted_iota(jnp.int32, shape, dim).astype(jnp.float32)


def _router_kernel(x_ref, rw_ref, pos_ref, texp_ref, gates_ref, aux_ref):
    xf = x_ref[...]
    logits = jax.lax.dot_general(
        xf, rw_ref[...], (((1,), (1,)), ((), ())),
        preferred_element_type=jnp.float32)
    m = jnp.max(logits, axis=1, keepdims=True)
    ex = jnp.exp(logits - m)
    probs = ex / jnp.sum(ex, axis=1, keepdims=True)

    eidx = _fiota((T, E), 1)
    v0 = jnp.max(probs, axis=1, keepdims=True)
    i0 = jnp.min(jnp.where(probs == v0, eidx, float(E)), axis=1,
                 keepdims=True)
    probs2 = jnp.where(eidx == i0, -1.0, probs)
    v1 = jnp.max(probs2, axis=1, keepdims=True)
    i1 = jnp.min(jnp.where(probs2 == v1, eidx, float(E)), axis=1,
                 keepdims=True)
    s = v0 + v1
    gates_ref[...] = jnp.concatenate([v0 / s, v1 / s], axis=1)

    M0 = (eidx == i0).astype(jnp.float32)
    M1 = (eidx == i1).astype(jnp.float32)
    rT = _fiota((T, T), 0)
    cT = _fiota((T, T), 1)
    LT = (cT < rT).astype(jnp.float32)
    Mcat = jnp.concatenate([M0, M1], axis=1)
    cscat = jax.lax.dot_general(LT, Mcat, (((1,), (0,)), ((), ())),
                                preferred_element_type=jnp.float32)
    cs0 = cscat[:, :E]
    cs1 = cscat[:, E:]

    count0 = jnp.sum(M0, axis=0, keepdims=True)
    counts = count0 + jnp.sum(M1, axis=0, keepdims=True)
    ac = jnp.ceil(counts / TILE) * TILE
    r8 = _fiota((E, E), 0)
    c8 = _fiota((E, E), 1)
    S8 = (r8 < c8).astype(jnp.float32)
    off = jax.lax.dot_general(ac, S8, (((1,), (0,)), ((), ())),
                              preferred_element_type=jnp.float32)

    pos0 = jnp.sum(M0 * (off + cs0), axis=1, keepdims=True)
    pos1 = jnp.sum(M1 * (off + count0 + cs1), axis=1, keepdims=True)
    pos_ref[...] = jnp.concatenate([pos0, pos1], axis=1).astype(jnp.int32)

    tstart = _fiota((NT, E), 0) * float(TILE)
    te = jnp.sum((jnp.broadcast_to(off, (NT, E)) <= tstart)
                 .astype(jnp.float32), axis=1, keepdims=True) - 1.0
    texp_ref[...] = te.astype(jnp.int32)

    f = counts / float(T)
    pmean = jnp.mean(probs, axis=0, keepdims=True)
    aux_ref[...] = jnp.sum(f * pmean, axis=1, keepdims=True) * (float(E) * 0.01)


def _router(xf, router_w):
    return pl.pallas_call(
        _router_kernel,
        out_shape=(
            jax.ShapeDtypeStruct((T, 2), jnp.int32),
            jax.ShapeDtypeStruct((NT, 1), jnp.int32),
            jax.ShapeDtypeStruct((T, 2), jnp.float32),
            jax.ShapeDtypeStruct((1, 1), jnp.float32),
        ),
    )(xf, router_w)


@functools.lru_cache(maxsize=None)
def _sc_kernels():
    mesh = plsc.VectorSubcoreMesh(
        core_axis_name="c", subcore_axis_name="s",
        num_cores=NC, num_subcores=NS)
    scratch = [
        pltpu.VMEM((NCH, CHS), jnp.int32),
        pltpu.VMEM((CHS, D), jnp.float32),
        pltpu.SemaphoreType.DMA,
    ]

    @functools.partial(
        pl.kernel,
        out_type=jax.ShapeDtypeStruct((P, D), jnp.float32),
        mesh=mesh, scratch_types=scratch)
    def sc_scatter_x(x_hbm, pos_hbm, xg_hbm, idx_v, rows_v, sem):
        wid = lax.axis_index("s") * NC + lax.axis_index("c")
        pltpu.sync_copy(pos_hbm.at[wid], idx_v)
        t0 = (wid * APW) % T

        def body(j, _):
            pltpu.sync_copy(x_hbm.at[pl.ds(t0 + j * CHS, CHS)], rows_v)
            pltpu.async_copy(rows_v, xg_hbm.at[idx_v.at[j]], sem).wait()
            return 0

        lax.fori_loop(0, NCH, body, 0, unroll=True)

    @functools.partial(
        pl.kernel,
        out_type=jax.ShapeDtypeStruct((A, D), jnp.float32),
        mesh=mesh, scratch_types=scratch)
    def sc_gather_y(yg_hbm, pos_hbm, yc_hbm, idx_v, rows_v, sem):
        wid = lax.axis_index("s") * NC + lax.axis_index("c")
        pltpu.sync_copy(pos_hbm.at[wid], idx_v)
        base = wid * APW

        def body(j, _):
            pltpu.async_copy(yg_hbm.at[idx_v.at[j]], rows_v, sem).wait()
            pltpu.sync_copy(rows_v, yc_hbm.at[pl.ds(base + j * CHS, CHS)])
            return 0

        lax.fori_loop(0, NCH, body, 0, unroll=True)

    return sc_scatter_x, sc_gather_y


def _group_kernel(texp_ref, xg_ref, wfc_ref, wproj_ref, yg_ref):
    rows = xg_ref[...].astype(jnp.bfloat16)
    h = jax.lax.dot_general(rows, wfc_ref[0].astype(jnp.bfloat16),
                            (((1,), (1,)), ((), ())),
                            preferred_element_type=jnp.float32)
    g = h[:, :F]
    u = h[:, F:]
    act = (g * jax.nn.sigmoid(g) * u).astype(jnp.bfloat16)
    yg_ref[...] = jax.lax.dot_general(
        act, wproj_ref[0].astype(jnp.bfloat16), (((1,), (1,)), ((), ())),
        preferred_element_type=jnp.float32)


def _grouped_mlp(xg, w_fc, w_proj, tile_expert):
    grid_spec = pltpu.PrefetchScalarGridSpec(
        num_scalar_prefetch=1,
        grid=(NT,),
        in_specs=[
            pl.BlockSpec((TILE, D), lambda i, te: (i, 0)),
            pl.BlockSpec((1, 2 * F, D), lambda i, te: (te[i], 0, 0)),
            pl.BlockSpec((1, D, F), lambda i, te: (te[i], 0, 0)),
        ],
        out_specs=pl.BlockSpec((TILE, D), lambda i, te: (i, 0)),
    )
    return pl.pallas_call(
        _group_kernel,
        grid_spec=grid_spec,
        out_shape=jax.ShapeDtypeStruct((P, D), jnp.float32),
    )(tile_expert, xg, w_fc, w_proj)


def _combine_kernel(y0_ref, y1_ref, gates_ref, out_ref):
    gts = gates_ref[...]
    out_ref[...] = (y0_ref[...] * gts[:, 0:1] + y1_ref[...] * gts[:, 1:2])


def _combine(yc, gates):
    blk = 256
    return pl.pallas_call(
        _combine_kernel,
        grid=(T // blk,),
        in_specs=[
            pl.BlockSpec((blk, D), lambda i: (i, 0)),
            pl.BlockSpec((blk, D), lambda i: (i + T // blk, 0)),
            pl.BlockSpec((blk, 2), lambda i: (i, 0)),
        ],
        out_specs=pl.BlockSpec((blk, D), lambda i: (i, 0)),
        out_shape=jax.ShapeDtypeStruct((T, D), jnp.float32),
    )(yc, yc, gates)


def kernel(x, router_w, w_fc, w_proj):
    b, s, d = x.shape
    xf = x.reshape(b * s, d)
    pos, tile_expert, gates, aux = _router(xf, router_w)

    sc_scatter_x, sc_gather_y = _sc_kernels()
    pos3 = pos.T.reshape(NW, NCH, CHS)
    xg = sc_scatter_x(xf, pos3)
    yg = _grouped_mlp(xg, w_fc, w_proj, tile_expert[:, 0])
    yc = sc_gather_y(yg, pos3)
    y = _combine(yc, gates).reshape(b, s, d)
    return (y, aux[0, 0])


# speedup vs baseline: 1.4237x; 1.0191x over previous
import functools

import jax
import jax.numpy as jnp
from jax import lax
from jax.experimental import pallas as pl
from jax.experimental.pallas import tpu as pltpu
from jax.experimental.pallas import tpu_sc as plsc

T = 2048
D = 2048
F = 1024
E = 8
K = 2
TILE = 256
A = T * K
P = A + E * TILE
NT = P // TILE
CB = 128
NCB = A // CB

NC = 2
NS = 16
NW = NC * NS
CHS = 16
APW = A // NW
NCH = APW // CHS


def _fiota(shape, dim):
    return lax.broadcasted_iota(jnp.int32, shape, dim).astype(jnp.float32)


def _router_kernel(x_ref, rw_ref, pos_ref, texp_ref, gates_ref, aux_ref):
    xf = x_ref[...]
    logits = jax.lax.dot_general(
        xf, rw_ref[...], (((1,), (1,)), ((), ())),
        preferred_element_type=jnp.float32)
    m = jnp.max(logits, axis=1, keepdims=True)
    ex = jnp.exp(logits - m)
    probs = ex / jnp.sum(ex, axis=1, keepdims=True)

    eidx = _fiota((T, E), 1)
    v0 = jnp.max(probs, axis=1, keepdims=True)
    i0 = jnp.min(jnp.where(probs == v0, eidx, float(E)), axis=1,
                 keepdims=True)
    probs2 = jnp.where(eidx == i0, -1.0, probs)
    v1 = jnp.max(probs2, axis=1, keepdims=True)
    i1 = jnp.min(jnp.where(probs2 == v1, eidx, float(E)), axis=1,
                 keepdims=True)
    s = v0 + v1
    gates_ref[...] = jnp.concatenate([v0 / s, v1 / s], axis=1)

    M0 = (eidx == i0).astype(jnp.float32)
    M1 = (eidx == i1).astype(jnp.float32)
    rT = _fiota((T, T), 0)
    cT = _fiota((T, T), 1)
    LT = (cT < rT).astype(jnp.float32)
    Mcat = jnp.concatenate([M0, M1], axis=1)
    cscat = jax.lax.dot_general(LT.astype(jnp.bfloat16),
                                Mcat.astype(jnp.bfloat16),
                                (((1,), (0,)), ((), ())),
                                preferred_element_type=jnp.float32)
    cs0 = cscat[:, :E]
    cs1 = cscat[:, E:]

    count0 = jnp.sum(M0, axis=0, keepdims=True)
    counts = count0 + jnp.sum(M1, axis=0, keepdims=True)
    ac = jnp.ceil(counts / TILE) * TILE
    r8 = _fiota((E, E), 0)
    c8 = _fiota((E, E), 1)
    S8 = (r8 < c8).astype(jnp.float32)
    off = jax.lax.dot_general(ac, S8, (((1,), (0,)), ((), ())),
                              preferred_element_type=jnp.float32)

    pos0 = jnp.sum(M0 * (off + cs0), axis=1, keepdims=True)
    pos1 = jnp.sum(M1 * (off + count0 + cs1), axis=1, keepdims=True)
    pos_ref[...] = jnp.concatenate([pos0, pos1], axis=1).astype(jnp.int32)

    tstart = _fiota((NT, E), 0) * float(TILE)
    te = jnp.sum((jnp.broadcast_to(off, (NT, E)) <= tstart)
                 .astype(jnp.float32), axis=1, keepdims=True) - 1.0
    texp_ref[...] = te.astype(jnp.int32)

    f = counts / float(T)
    pmean = jnp.mean(probs, axis=0, keepdims=True)
    aux_ref[...] = jnp.sum(f * pmean, axis=1, keepdims=True) * (float(E) * 0.01)


def _router(xf, router_w):
    return pl.pallas_call(
        _router_kernel,
        out_shape=(
            jax.ShapeDtypeStruct((T, 2), jnp.int32),
            jax.ShapeDtypeStruct((NT, 1), jnp.int32),
            jax.ShapeDtypeStruct((T, 2), jnp.float32),
            jax.ShapeDtypeStruct((1, 1), jnp.float32),
        ),
    )(xf, router_w)


@functools.lru_cache(maxsize=None)
def _sc_kernels():
    mesh = plsc.VectorSubcoreMesh(
        core_axis_name="c", subcore_axis_name="s",
        num_cores=NC, num_subcores=NS)
    scratch = [
        pltpu.VMEM((NCH, CHS), jnp.int32),
        pltpu.VMEM((2, CHS, D), jnp.float32),
        pltpu.SemaphoreType.DMA,
        pltpu.SemaphoreType.DMA,
    ]

    @functools.partial(
        pl.kernel,
        out_type=jax.ShapeDtypeStruct((P, D), jnp.float32),
        mesh=mesh, scratch_types=scratch)
    def sc_scatter_x(x_hbm, pos_hbm, xg_hbm, idx_v, rows_v, sem_r, sem_w):
        wid = lax.axis_index("s") * NC + lax.axis_index("c")
        pltpu.sync_copy(pos_hbm.at[wid], idx_v)
        t0 = (wid * APW) % T

        rd = [None] * NCH
        rd[0] = pltpu.async_copy(
            x_hbm.at[pl.ds(t0, CHS)], rows_v.at[0], sem_r)
        for j in range(NCH):
            rd[j].wait()
            if j + 1 < NCH:
                rd[j + 1] = pltpu.async_copy(
                    x_hbm.at[pl.ds(t0 + (j + 1) * CHS, CHS)],
                    rows_v.at[(j + 1) % 2], sem_r)
            pltpu.async_copy(
                rows_v.at[j % 2], xg_hbm.at[idx_v.at[j]], sem_w).wait()

    @functools.partial(
        pl.kernel,
        out_type=jax.ShapeDtypeStruct((A, D), jnp.float32),
        mesh=mesh, scratch_types=scratch)
    def sc_gather_y(yg_hbm, pos_hbm, yc_hbm, idx_v, rows_v, sem_r, sem_w):
        wid = lax.axis_index("s") * NC + lax.axis_index("c")
        pltpu.sync_copy(pos_hbm.at[wid], idx_v)
        base = wid * APW

        rd = [None] * NCH
        rd[0] = pltpu.async_copy(yg_hbm.at[idx_v.at[0]], rows_v.at[0], sem_r)
        for j in range(NCH):
            rd[j].wait()
            if j + 1 < NCH:
                rd[j + 1] = pltpu.async_copy(
                    yg_hbm.at[idx_v.at[j + 1]], rows_v.at[(j + 1) % 2], sem_r)
            pltpu.async_copy(
                rows_v.at[j % 2], yc_hbm.at[pl.ds(base + j * CHS, CHS)],
                sem_w).wait()

    return sc_scatter_x, sc_gather_y


def _group_kernel(texp_ref, xg_ref, wfc_ref, wproj_ref, yg_ref):
    rows = xg_ref[...].astype(jnp.bfloat16)
    h = jax.lax.dot_general(rows, wfc_ref[0].astype(jnp.bfloat16),
                            (((1,), (1,)), ((), ())),
                            preferred_element_type=jnp.float32)
    g = h[:, :F]
    u = h[:, F:]
    act = (g * jax.nn.sigmoid(g) * u).astype(jnp.bfloat16)
    yg_ref[...] = jax.lax.dot_general(
        act, wproj_ref[0].astype(jnp.bfloat16), (((1,), (1,)), ((), ())),
        preferred_element_type=jnp.float32)


def _grouped_mlp(xg, w_fc, w_proj, tile_expert):
    grid_spec = pltpu.PrefetchScalarGridSpec(
        num_scalar_prefetch=1,
        grid=(NT,),
        in_specs=[
            pl.BlockSpec((TILE, D), lambda i, te: (i, 0)),
            pl.BlockSpec((1, 2 * F, D), lambda i, te: (te[i], 0, 0)),
            pl.BlockSpec((1, D, F), lambda i, te: (te[i], 0, 0)),
        ],
        out_specs=pl.BlockSpec((TILE, D), lambda i, te: (i, 0)),
    )
    return pl.pallas_call(
        _group_kernel,
        grid_spec=grid_spec,
        out_shape=jax.ShapeDtypeStruct((P, D), jnp.float32),
    )(tile_expert, xg, w_fc, w_proj)


def _combine_kernel(y0_ref, y1_ref, gates_ref, out_ref):
    gts = gates_ref[...]
    out_ref[...] = (y0_ref[...] * gts[:, 0:1] + y1_ref[...] * gts[:, 1:2])


def _combine(yc, gates):
    blk = 256
    return pl.pallas_call(
        _combine_kernel,
        grid=(T // blk,),
        in_specs=[
            pl.BlockSpec((blk, D), lambda i: (i, 0)),
            pl.BlockSpec((blk, D), lambda i: (i + T // blk, 0)),
            pl.BlockSpec((blk, 2), lambda i: (i, 0)),
        ],
        out_specs=pl.BlockSpec((blk, D), lambda i: (i, 0)),
        out_shape=jax.ShapeDtypeStruct((T, D), jnp.float32),
    )(yc, yc, gates)


def kernel(x, router_w, w_fc, w_proj):
    b, s, d = x.shape
    xf = x.reshape(b * s, d)
    pos, tile_expert, gates, aux = _router(xf, router_w)

    sc_scatter_x, sc_gather_y = _sc_kernels()
    pos3 = pos.T.reshape(NW, NCH, CHS)
    xg = sc_scatter_x(xf, pos3)
    yg = _grouped_mlp(xg, w_fc, w_proj, tile_expert[:, 0])
    yc = sc_gather_y(yg, pos3)
    y = _combine(yc, gates).reshape(b, s, d)
    return (y, aux[0, 0])
